# baseline (speedup 1.0000x reference)
import functools

import jax
import jax.numpy as jnp
from jax import lax
from jax.experimental import pallas as pl
from jax.experimental.pallas import tpu as pltpu
from jax.experimental.pallas import tpu_sc as plsc

R = 128
C = 32768
R_TC = 96
R_SC = R - R_TC
NCORE = 2
NSUB = 16
L = 16
NW = NCORE * NSUB
RPW = max(1, R_SC // NW)
NACC = 8
VPB = L * NACC
NIT = C // VPB
BIG = 0x7FFFFFFF
TCB = 8

_sc_scratch = (
    [pltpu.VMEM((C,), jnp.float32) for _ in range(2)]
    + [pltpu.VMEM((L,), jnp.int32)]
    + [pltpu.SemaphoreType.DMA, pltpu.SemaphoreType.DMA]
)


def _sc_body(x_hbm, out_hbm, buf0, buf1, res_v, sem0, sem1):
    wid = lax.axis_index("s") * NCORE + lax.axis_index("c")
    row0 = R_TC + wid * RPW
    bufs = (buf0, buf1)
    sems = (sem0, sem1)
    lanes = lax.iota(jnp.int32, L)

    pltpu.make_async_copy(x_hbm.at[row0], buf0, sem0).start()
    if RPW > 1:
        pltpu.make_async_copy(x_hbm.at[row0 + 1], buf1, sem1).start()

    resvec = jnp.zeros((L,), jnp.int32)
    for j in range(RPW):
        buf = bufs[j % 2]
        sem = sems[j % 2]
        pltpu.make_async_copy(x_hbm.at[row0 + j], buf, sem).wait()

        neg = jnp.full((L,), -jnp.inf, jnp.float32)
        init = (
            tuple(neg for _ in range(NACC)),
            tuple(jnp.zeros((L,), jnp.int32) for _ in range(NACC)),
            tuple(lanes + a * L for a in range(NACC)),
        )

        @plsc.parallel_loop(0, NIT, step=1, unroll=2, carry=init)
        def loop_out(it, carry, buf=buf):
            best, bidx, idx = carry
            base = it * VPB
            nb = []
            ni = []
            nx = []
            for a in range(NACC):
                v = buf[pl.ds(base + a * L, L)]
                m = v > best[a]
                nb.append(jnp.where(m, v, best[a]))
                ni.append(jnp.where(m, idx[a], bidx[a]))
                nx.append(idx[a] + VPB)
            return tuple(nb), tuple(ni), tuple(nx)

        best, bidx, _ = loop_out

        if j + 2 < RPW:
            pltpu.make_async_copy(x_hbm.at[row0 + j + 2], buf, sem).start()

        cb, ci = best[0], bidx[0]
        for a in range(1, NACC):
            take = (best[a] > cb) | ((best[a] == cb) & (bidx[a] < ci))
            cb = jnp.where(take, best[a], cb)
            ci = jnp.where(take, bidx[a], ci)

        rowmax = cb
        for sh in (8, 4, 2, 1):
            rot = (lanes + sh) & (L - 1)
            rowmax = jnp.maximum(
                rowmax, rowmax.at[rot].get(mode="promise_in_bounds")
            )
        cand = jnp.where(cb == rowmax, ci, jnp.full((L,), BIG, jnp.int32))
        for sh in (8, 4, 2, 1):
            rot = (lanes + sh) & (L - 1)
            cand = jnp.minimum(
                cand, cand.at[rot].get(mode="promise_in_bounds")
            )
        resvec = jnp.where(lanes == j, cand, resvec)

    res_v[...] = resvec
    pltpu.sync_copy(res_v, out_hbm.at[pl.ds(wid * L, L)])


@functools.cache
def _get_sc_kernel():
    mesh = plsc.VectorSubcoreMesh(
        core_axis_name="c",
        subcore_axis_name="s",
        num_cores=NCORE,
        num_subcores=NSUB,
    )
    return pl.kernel(
        _sc_body,
        out_type=jax.ShapeDtypeStruct((NW * L,), jnp.int32),
        mesh=mesh,
        scratch_types=_sc_scratch,
        compiler_params=pltpu.CompilerParams(skip_device_barrier=True),
    )


TC_NRB = R_TC // TCB
TC_NACC = 8
TC_NT = C // 128


def _tc_body(x_ref, o_ref):
    lane = lax.broadcasted_iota(jnp.int32, (TCB, 128), 1)
    best = [jnp.full((TCB, 128), -jnp.inf, jnp.float32) for _ in range(TC_NACC)]
    bidx = [jnp.zeros((TCB, 128), jnp.int32) for _ in range(TC_NACC)]
    for t in range(TC_NT):
        a = t % TC_NACC
        x = x_ref[:, pl.ds(t * 128, 128)]
        idx = lane + (t * 128)
        m = x > best[a]
        best[a] = jnp.where(m, x, best[a])
        bidx[a] = jnp.where(m, idx, bidx[a])
    cb, ci = best[0], bidx[0]
    for a in range(1, TC_NACC):
        take = (best[a] > cb) | ((best[a] == cb) & (bidx[a] < ci))
        cb = jnp.where(take, best[a], cb)
        ci = jnp.where(take, bidx[a], ci)
    mx = jnp.max(cb, axis=1, keepdims=True)
    cand = jnp.where(cb == mx, ci, BIG)
    o_ref[...] = jnp.min(cand, axis=1)[None, None, :]


def _tc_argmax(x):
    x = pltpu.with_memory_space_constraint(x, pltpu.MemorySpace.HBM)
    return pl.pallas_call(
        _tc_body,
        grid=(TC_NRB,),
        in_specs=[pl.BlockSpec((TCB, C), lambda i: (i, 0))],
        out_specs=pl.BlockSpec((1, 1, TCB), lambda i: (i, 0, 0)),
        out_shape=jax.ShapeDtypeStruct((TC_NRB, 1, TCB), jnp.int32),
    )(x)


def kernel(x):
    tc_out = _tc_argmax(x)
    tc_rows = tc_out.reshape(R_TC)
    if R_SC:
        sc_out = _get_sc_kernel()(x)
        sc_rows = sc_out.reshape(NW, L)[:, :RPW].reshape(R_SC)
        tc_rows = jnp.concatenate([tc_rows, sc_rows])
    return tc_rows.astype(jnp.int64)


# speedup vs baseline: 2.8390x; 2.8390x over previous
import functools

import jax
import jax.numpy as jnp
from jax import lax
from jax.experimental import pallas as pl
from jax.experimental.pallas import tpu as pltpu
from jax.experimental.pallas import tpu_sc as plsc

R = 128
C = 32768
R_TC = 128
R_SC = R - R_TC
NCORE = 2
NSUB = 16
L = 16
NW = NCORE * NSUB
RPW = max(1, R_SC // NW)
NACC = 8
VPB = L * NACC
NIT = C // VPB
BIG = 0x7FFFFFFF
TCB = 8

_sc_scratch = (
    [pltpu.VMEM((C,), jnp.float32) for _ in range(2)]
    + [pltpu.VMEM((L,), jnp.int32)]
    + [pltpu.SemaphoreType.DMA, pltpu.SemaphoreType.DMA]
)


def _sc_body(x_hbm, out_hbm, buf0, buf1, res_v, sem0, sem1):
    wid = lax.axis_index("s") * NCORE + lax.axis_index("c")
    row0 = R_TC + wid * RPW
    bufs = (buf0, buf1)
    sems = (sem0, sem1)
    lanes = lax.iota(jnp.int32, L)

    pltpu.make_async_copy(x_hbm.at[row0], buf0, sem0).start()
    if RPW > 1:
        pltpu.make_async_copy(x_hbm.at[row0 + 1], buf1, sem1).start()

    resvec = jnp.zeros((L,), jnp.int32)
    for j in range(RPW):
        buf = bufs[j % 2]
        sem = sems[j % 2]
        pltpu.make_async_copy(x_hbm.at[row0 + j], buf, sem).wait()

        neg = jnp.full((L,), -jnp.inf, jnp.float32)
        init = (
            tuple(neg for _ in range(NACC)),
            tuple(jnp.zeros((L,), jnp.int32) for _ in range(NACC)),
            tuple(lanes + a * L for a in range(NACC)),
        )

        @plsc.parallel_loop(0, NIT, step=1, unroll=2, carry=init)
        def loop_out(it, carry, buf=buf):
            best, bidx, idx = carry
            base = it * VPB
            nb = []
            ni = []
            nx = []
            for a in range(NACC):
                v = buf[pl.ds(base + a * L, L)]
                m = v > best[a]
                nb.append(jnp.where(m, v, best[a]))
                ni.append(jnp.where(m, idx[a], bidx[a]))
                nx.append(idx[a] + VPB)
            return tuple(nb), tuple(ni), tuple(nx)

        best, bidx, _ = loop_out

        if j + 2 < RPW:
            pltpu.make_async_copy(x_hbm.at[row0 + j + 2], buf, sem).start()

        cb, ci = best[0], bidx[0]
        for a in range(1, NACC):
            take = (best[a] > cb) | ((best[a] == cb) & (bidx[a] < ci))
            cb = jnp.where(take, best[a], cb)
            ci = jnp.where(take, bidx[a], ci)

        rowmax = cb
        for sh in (8, 4, 2, 1):
            rot = (lanes + sh) & (L - 1)
            rowmax = jnp.maximum(
                rowmax, rowmax.at[rot].get(mode="promise_in_bounds")
            )
        cand = jnp.where(cb == rowmax, ci, jnp.full((L,), BIG, jnp.int32))
        for sh in (8, 4, 2, 1):
            rot = (lanes + sh) & (L - 1)
            cand = jnp.minimum(
                cand, cand.at[rot].get(mode="promise_in_bounds")
            )
        resvec = jnp.where(lanes == j, cand, resvec)

    res_v[...] = resvec
    pltpu.sync_copy(res_v, out_hbm.at[pl.ds(wid * L, L)])


@functools.cache
def _get_sc_kernel():
    mesh = plsc.VectorSubcoreMesh(
        core_axis_name="c",
        subcore_axis_name="s",
        num_cores=NCORE,
        num_subcores=NSUB,
    )
    return pl.kernel(
        _sc_body,
        out_type=jax.ShapeDtypeStruct((NW * L,), jnp.int32),
        mesh=mesh,
        scratch_types=_sc_scratch,
        compiler_params=pltpu.CompilerParams(skip_device_barrier=True),
    )


TC_NRB = R_TC // TCB
TC_NACC = 8
TC_NT = C // 128
TC_NBUF = 5


def _tc_body(x_hbm, o_ref, *rest):
    bufs = rest[:TC_NBUF]
    sems = rest[TC_NBUF:]

    def blk_copy(rb):
        return pltpu.make_async_copy(
            x_hbm.at[pl.ds(rb * TCB, TCB)], bufs[rb % TC_NBUF], sems[rb % TC_NBUF]
        )

    for rb in range(min(TC_NBUF, TC_NRB)):
        blk_copy(rb).start()

    lane = lax.broadcasted_iota(jnp.int32, (TCB, 128), 1)
    for rb in range(TC_NRB):
        blk_copy(rb).wait()
        buf = bufs[rb % TC_NBUF]
        best = [jnp.full((TCB, 128), -jnp.inf, jnp.float32) for _ in range(TC_NACC)]
        bidx = [jnp.zeros((TCB, 128), jnp.int32) for _ in range(TC_NACC)]
        for t in range(TC_NT):
            a = t % TC_NACC
            x = buf[:, pl.ds(t * 128, 128)]
            idx = lane + (t * 128)
            m = x > best[a]
            best[a] = jnp.where(m, x, best[a])
            bidx[a] = jnp.where(m, idx, bidx[a])
        if rb + TC_NBUF < TC_NRB:
            blk_copy(rb + TC_NBUF).start()
        cb, ci = best[0], bidx[0]
        for a in range(1, TC_NACC):
            take = (best[a] > cb) | ((best[a] == cb) & (bidx[a] < ci))
            cb = jnp.where(take, best[a], cb)
            ci = jnp.where(take, bidx[a], ci)
        mx = jnp.max(cb, axis=1, keepdims=True)
        cand = jnp.where(cb == mx, ci, BIG)
        o_ref[rb, 0] = jnp.min(cand, axis=1)


def _tc_argmax(x):
    x = pltpu.with_memory_space_constraint(x, pltpu.MemorySpace.HBM)
    return pl.pallas_call(
        _tc_body,
        in_specs=[pl.BlockSpec(memory_space=pl.ANY)],
        out_shape=jax.ShapeDtypeStruct((TC_NRB, 1, TCB), jnp.int32),
        scratch_shapes=(
            [pltpu.VMEM((TCB, C), jnp.float32) for _ in range(TC_NBUF)]
            + [pltpu.SemaphoreType.DMA for _ in range(TC_NBUF)]
        ),
    )(x)


def kernel(x):
    tc_out = _tc_argmax(x)
    tc_rows = tc_out.reshape(R_TC)
    if R_SC:
        sc_out = _get_sc_kernel()(x)
        sc_rows = sc_out.reshape(NW, L)[:, :RPW].reshape(R_SC)
        tc_rows = jnp.concatenate([tc_rows, sc_rows])
    return tc_rows.astype(jnp.int64)


# speedup vs baseline: 2.8450x; 1.0021x over previous
import functools

import jax
import jax.numpy as jnp
from jax import lax
from jax.experimental import pallas as pl
from jax.experimental.pallas import tpu as pltpu
from jax.experimental.pallas import tpu_sc as plsc

R = 128
C = 32768
R_TC = 128
R_SC = R - R_TC
NCORE = 2
NSUB = 16
L = 16
NW = NCORE * NSUB
RPW = max(1, R_SC // NW)
NACC = 8
VPB = L * NACC
NIT = C // VPB
BIG = 0x7FFFFFFF
TCB = 8

_sc_scratch = (
    [pltpu.VMEM((C,), jnp.float32) for _ in range(2)]
    + [pltpu.VMEM((L,), jnp.int32)]
    + [pltpu.SemaphoreType.DMA, pltpu.SemaphoreType.DMA]
)


def _sc_body(x_hbm, out_hbm, buf0, buf1, res_v, sem0, sem1):
    wid = lax.axis_index("s") * NCORE + lax.axis_index("c")
    row0 = R_TC + wid * RPW
    bufs = (buf0, buf1)
    sems = (sem0, sem1)
    lanes = lax.iota(jnp.int32, L)

    pltpu.make_async_copy(x_hbm.at[row0], buf0, sem0).start()
    if RPW > 1:
        pltpu.make_async_copy(x_hbm.at[row0 + 1], buf1, sem1).start()

    resvec = jnp.zeros((L,), jnp.int32)
    for j in range(RPW):
        buf = bufs[j % 2]
        sem = sems[j % 2]
        pltpu.make_async_copy(x_hbm.at[row0 + j], buf, sem).wait()

        neg = jnp.full((L,), -jnp.inf, jnp.float32)
        init = (
            tuple(neg for _ in range(NACC)),
            tuple(jnp.zeros((L,), jnp.int32) for _ in range(NACC)),
            tuple(lanes + a * L for a in range(NACC)),
        )

        @plsc.parallel_loop(0, NIT, step=1, unroll=2, carry=init)
        def loop_out(it, carry, buf=buf):
            best, bidx, idx = carry
            base = it * VPB
            nb = []
            ni = []
            nx = []
            for a in range(NACC):
                v = buf[pl.ds(base + a * L, L)]
                m = v > best[a]
                nb.append(jnp.where(m, v, best[a]))
                ni.append(jnp.where(m, idx[a], bidx[a]))
                nx.append(idx[a] + VPB)
            return tuple(nb), tuple(ni), tuple(nx)

        best, bidx, _ = loop_out

        if j + 2 < RPW:
            pltpu.make_async_copy(x_hbm.at[row0 + j + 2], buf, sem).start()

        cb, ci = best[0], bidx[0]
        for a in range(1, NACC):
            take = (best[a] > cb) | ((best[a] == cb) & (bidx[a] < ci))
            cb = jnp.where(take, best[a], cb)
            ci = jnp.where(take, bidx[a], ci)

        rowmax = cb
        for sh in (8, 4, 2, 1):
            rot = (lanes + sh) & (L - 1)
            rowmax = jnp.maximum(
                rowmax, rowmax.at[rot].get(mode="promise_in_bounds")
            )
        cand = jnp.where(cb == rowmax, ci, jnp.full((L,), BIG, jnp.int32))
        for sh in (8, 4, 2, 1):
            rot = (lanes + sh) & (L - 1)
            cand = jnp.minimum(
                cand, cand.at[rot].get(mode="promise_in_bounds")
            )
        resvec = jnp.where(lanes == j, cand, resvec)

    res_v[...] = resvec
    pltpu.sync_copy(res_v, out_hbm.at[pl.ds(wid * L, L)])


@functools.cache
def _get_sc_kernel():
    mesh = plsc.VectorSubcoreMesh(
        core_axis_name="c",
        subcore_axis_name="s",
        num_cores=NCORE,
        num_subcores=NSUB,
    )
    return pl.kernel(
        _sc_body,
        out_type=jax.ShapeDtypeStruct((NW * L,), jnp.int32),
        mesh=mesh,
        scratch_types=_sc_scratch,
        compiler_params=pltpu.CompilerParams(skip_device_barrier=True),
    )


TC_NRB = R_TC // TCB
TC_NACC = 8
TC_NT = C // 128
TC_NBUF = 8


def _tc_body(x_hbm, o_ref, *rest):
    bufs = rest[:TC_NBUF]
    sems = rest[TC_NBUF:]

    def blk_copy(rb):
        return pltpu.make_async_copy(
            x_hbm.at[pl.ds(rb * TCB, TCB)], bufs[rb % TC_NBUF], sems[rb % TC_NBUF]
        )

    for rb in range(min(TC_NBUF, TC_NRB)):
        blk_copy(rb).start()

    lane = lax.broadcasted_iota(jnp.int32, (TCB, 128), 1)
    for rb in range(TC_NRB):
        blk_copy(rb).wait()
        buf = bufs[rb % TC_NBUF]
        best = [jnp.full((TCB, 128), -jnp.inf, jnp.float32) for _ in range(TC_NACC)]
        bidx = [jnp.zeros((TCB, 128), jnp.int32) for _ in range(TC_NACC)]
        for t in range(TC_NT):
            a = t % TC_NACC
            x = buf[:, pl.ds(t * 128, 128)]
            idx = lane + (t * 128)
            m = x > best[a]
            best[a] = jnp.where(m, x, best[a])
            bidx[a] = jnp.where(m, idx, bidx[a])
        if rb + TC_NBUF < TC_NRB:
            blk_copy(rb + TC_NBUF).start()
        cb, ci = best[0], bidx[0]
        for a in range(1, TC_NACC):
            take = (best[a] > cb) | ((best[a] == cb) & (bidx[a] < ci))
            cb = jnp.where(take, best[a], cb)
            ci = jnp.where(take, bidx[a], ci)
        mx = jnp.max(cb, axis=1, keepdims=True)
        cand = jnp.where(cb == mx, ci, BIG)
        o_ref[rb, 0] = jnp.min(cand, axis=1)


def _tc_argmax(x):
    x = pltpu.with_memory_space_constraint(x, pltpu.MemorySpace.HBM)
    return pl.pallas_call(
        _tc_body,
        in_specs=[pl.BlockSpec(memory_space=pl.ANY)],
        out_shape=jax.ShapeDtypeStruct((TC_NRB, 1, TCB), jnp.int32),
        scratch_shapes=(
            [pltpu.VMEM((TCB, C), jnp.float32) for _ in range(TC_NBUF)]
            + [pltpu.SemaphoreType.DMA for _ in range(TC_NBUF)]
        ),
    )(x)


def kernel(x):
    tc_out = _tc_argmax(x)
    tc_rows = tc_out.reshape(R_TC)
    if R_SC:
        sc_out = _get_sc_kernel()(x)
        sc_rows = sc_out.reshape(NW, L)[:, :RPW].reshape(R_SC)
        tc_rows = jnp.concatenate([tc_rows, sc_rows])
    return tc_rows.astype(jnp.int64)
